# baseline (speedup 1.0000x reference)
import functools

import jax
import jax.numpy as jnp
from jax import lax
from jax.experimental import pallas as pl
from jax.experimental.pallas import tpu as pltpu
from jax.experimental.pallas import tpu_sc as plsc

N = 4096
B = 4
S = 2048
ROWS = B * S
NC = 2
NS = 16
NW = NC * NS
ROWS_PER_W = ROWS // NW
WPB = S // ROWS_PER_W
K = 4
CHUNKS = ROWS_PER_W // K
NBI = 4
NBO = 2
L = 16


def _sc_permute(x, perm):
    mesh = plsc.VectorSubcoreMesh(core_axis_name="c", subcore_axis_name="s")

    @functools.partial(
        pl.kernel,
        mesh=mesh,
        out_type=jax.ShapeDtypeStruct((B, S, N), jnp.float32),
        compiler_params=pltpu.CompilerParams(needs_layout_passes=False),
        scratch_types=[
            pltpu.VMEM((N,), jnp.int32),
            *[pltpu.VMEM((K, N), jnp.float32) for _ in range(NBI + NBO)],
            *[pltpu.SemaphoreType.DMA for _ in range(NBI + NBO)],
        ],
    )
    def k(x_hbm, perm_hbm, out_hbm, perm_v, *bufs_and_sems):
        ins = list(bufs_and_sems[:NBI])
        outs = list(bufs_and_sems[NBI : NBI + NBO])
        sis = list(bufs_and_sems[NBI + NBO : 2 * NBI + NBO])
        sos = list(bufs_and_sems[2 * NBI + NBO : 2 * NBI + 2 * NBO])

        wid = lax.axis_index("s") * NC + lax.axis_index("c")
        pltpu.sync_copy(perm_hbm, perm_v)
        batch = wid // WPB
        row0 = (wid % WPB) * ROWS_PER_W

        def start_in(b, c):
            pltpu.async_copy(
                x_hbm.at[batch, pl.ds(row0 + c * K, K)], ins[b], sis[b]
            )

        for b in range(NBI):
            start_in(b, b)

        def outer(g, _):
            for i in range(NBI):
                c = g * NBI + i
                bi = i
                bo = i % NBO
                pltpu.make_async_copy(
                    x_hbm.at[0, pl.ds(0, K)], ins[bi], sis[bi]
                ).wait()

                @pl.when(c >= NBO)
                def _():
                    pltpu.make_async_copy(
                        outs[bo], out_hbm.at[0, pl.ds(0, K)], sos[bo]
                    ).wait()

                in_b = ins[bi]
                out_b = outs[bo]

                @plsc.parallel_loop(0, N // L, unroll=16)
                def col_body(j):
                    idx = perm_v[pl.ds(j * L, L)]
                    for r in range(K):
                        rows = jnp.full((L,), r, dtype=jnp.int32)
                        out_b[r, pl.ds(j * L, L)] = plsc.load_gather(
                            in_b, [rows, idx]
                        )

                pltpu.async_copy(
                    outs[bo], out_hbm.at[batch, pl.ds(row0 + c * K, K)],
                    sos[bo],
                )

                @pl.when(c + NBI < CHUNKS)
                def _():
                    start_in(bi, c + NBI)

            return 0

        lax.fori_loop(0, CHUNKS // NBI, outer, 0)
        for b in range(NBO):
            pltpu.make_async_copy(
                outs[b], out_hbm.at[0, pl.ds(0, K)], sos[b]
            ).wait()

    return k(x, perm)


def kernel(input, permutation):
    perm = permutation.astype(jnp.int32)
    return _sc_permute(input, perm)


# speedup vs baseline: 1.0182x; 1.0182x over previous
import functools

import jax
import jax.numpy as jnp
from jax import lax
from jax.experimental import pallas as pl
from jax.experimental.pallas import tpu as pltpu
from jax.experimental.pallas import tpu_sc as plsc

N = 4096
B = 4
S = 2048
ROWS = B * S
NC = 2
NS = 16
NW = NC * NS
ROWS_PER_W = ROWS // NW
WPB = S // ROWS_PER_W
K = 8
H = K // 2
CHUNKS = ROWS_PER_W // K
NBI = 2
L = 16


def _sc_permute(x, perm):
    mesh = plsc.VectorSubcoreMesh(core_axis_name="c", subcore_axis_name="s")

    @functools.partial(
        pl.kernel,
        mesh=mesh,
        out_type=jax.ShapeDtypeStruct((B, S, N), jnp.float32),
        compiler_params=pltpu.CompilerParams(needs_layout_passes=False),
        scratch_types=[
            pltpu.VMEM((N,), jnp.int32),
            *[pltpu.VMEM((K, N), jnp.float32) for _ in range(NBI)],
            *[pltpu.VMEM((H, N), jnp.float32) for _ in range(2)],
            *[pltpu.SemaphoreType.DMA for _ in range(NBI + 2)],
        ],
    )
    def k(x_hbm, perm_hbm, out_hbm, perm_v, in0, in1, out0, out1,
          si0, si1, so0, so1):
        ins = [in0, in1]
        outs = [out0, out1]
        sis = [si0, si1]
        sos = [so0, so1]

        wid = lax.axis_index("s") * NC + lax.axis_index("c")
        pltpu.sync_copy(perm_hbm, perm_v)
        batch = wid // WPB
        row0 = (wid % WPB) * ROWS_PER_W

        def start_in(b, c):
            pltpu.async_copy(
                x_hbm.at[batch, pl.ds(row0 + c * K, K)], ins[b], sis[b]
            )

        for b in range(NBI):
            start_in(b, b)

        def outer(g, _):
            for i in range(NBI):
                c = g * NBI + i
                bi = i
                pltpu.make_async_copy(
                    x_hbm.at[0, pl.ds(0, K)], ins[bi], sis[bi]
                ).wait()
                in_b = ins[bi]

                for h in range(2):
                    @pl.when(c >= 1)
                    def _():
                        pltpu.make_async_copy(
                            outs[h], out_hbm.at[0, pl.ds(0, H)], sos[h]
                        ).wait()

                    out_b = outs[h]

                    @plsc.parallel_loop(0, N // L, unroll=8)
                    def col_body(j):
                        idx = perm_v[pl.ds(j * L, L)]
                        for r in range(H):
                            rows = jnp.full((L,), h * H + r, dtype=jnp.int32)
                            out_b[r, pl.ds(j * L, L)] = plsc.load_gather(
                                in_b, [rows, idx]
                            )

                    pltpu.async_copy(
                        outs[h],
                        out_hbm.at[batch, pl.ds(row0 + c * K + h * H, H)],
                        sos[h],
                    )

                @pl.when(c + NBI < CHUNKS)
                def _():
                    start_in(bi, c + NBI)

            return 0

        lax.fori_loop(0, CHUNKS // NBI, outer, 0)
        for h in range(2):
            pltpu.make_async_copy(
                outs[h], out_hbm.at[0, pl.ds(0, H)], sos[h]
            ).wait()

    return k(x, perm)


def kernel(input, permutation):
    perm = permutation.astype(jnp.int32)
    return _sc_permute(input, perm)
